# baseline (speedup 1.0000x reference)
import functools

import jax
import jax.numpy as jnp
from jax import lax
from jax.experimental import pallas as pl
from jax.experimental.pallas import tpu as pltpu, tpu_sc as plsc

NUM_TILES = 32
TILES_PER_SC = 16
C = 128
FRAC0 = 0.374


def _deg_kernel_fn(n_pad, ch0, ch1, rows_per_tile):
  mesh = plsc.VectorSubcoreMesh(core_axis_name="c", subcore_axis_name="s")
  chm = max(ch0, ch1)

  @functools.partial(
      pl.kernel,
      out_type=jax.ShapeDtypeStruct((2, n_pad), jnp.float32),
      mesh=mesh,
      scratch_types=[
          pltpu.VMEM((chm, C), jnp.int32),
          pltpu.VMEM((C,), jnp.float32),
          pltpu.VMEM((rows_per_tile,), jnp.float32),
          pltpu.VMEM_SHARED((n_pad,), jnp.float32),
      ],
  )
  def deg_kernel(dst_hbm, out_hbm, idx_v, ones_v, zbuf_v, acc):
    cid = lax.axis_index("c")
    sid = lax.axis_index("s")

    def set_ones(i, _):
      ones_v[pl.ds(i * 16, 16)] = jnp.ones((16,), jnp.float32)
      return 0

    lax.fori_loop(0, C // 16, set_ones, 0)

    def set_zero(i, _):
      zbuf_v[pl.ds(i * 16, 16)] = jnp.zeros((16,), jnp.float32)
      return 0

    lax.fori_loop(0, rows_per_tile // 16, set_zero, 0)
    pltpu.sync_copy(zbuf_v, acc.at[pl.ds(sid * rows_per_tile, rows_per_tile)])
    plsc.subcore_barrier()

    def run(base, count):
      pltpu.sync_copy(dst_hbm.at[pl.ds(base, count)],
                      idx_v.at[pl.ds(0, count)])

      def body(j, _):
        pltpu.sync_copy(ones_v, acc.at[idx_v.at[j]], add=True)
        return 0

      lax.fori_loop(0, count, body, 0)

    @pl.when(cid == 0)
    def _():
      run(sid * ch0, ch0)

    @pl.when(cid == 1)
    def _():
      run(TILES_PER_SC * ch0 + sid * ch1, ch1)

    plsc.subcore_barrier()
    sl = pl.ds(sid * rows_per_tile, rows_per_tile)
    pltpu.sync_copy(acc.at[sl], out_hbm.at[cid, sl])

  return deg_kernel


def _agg_kernel_fn(n_pad, d, ch0, ch1, rows_per_tile):
  mesh = plsc.VectorSubcoreMesh(core_axis_name="c", subcore_axis_name="s")
  chm = max(ch0, ch1)

  @functools.partial(
      pl.kernel,
      out_type=jax.ShapeDtypeStruct((2, n_pad, d), jnp.float32),
      mesh=mesh,
      scratch_types=[
          pltpu.VMEM((chm, C), jnp.int32),
          pltpu.VMEM((chm, C), jnp.int32),
          pltpu.VMEM((C, d), jnp.float32),
          pltpu.VMEM_SHARED((n_pad, d), jnp.float32),
          pltpu.SemaphoreType.DMA,
      ],
  )
  def agg_kernel(src_hbm, dst_hbm, g_hbm, out_hbm, src_v, dst_v, rows_v, acc,
                 sem):
    cid = lax.axis_index("c")
    sid = lax.axis_index("s")

    def zero_row(i, _):
      def zero_chunk(k, _2):
        rows_v[i, pl.ds(k * 16, 16)] = jnp.zeros((16,), jnp.float32)
        return 0

      lax.fori_loop(0, d // 16, zero_chunk, 0)
      return 0

    lax.fori_loop(0, C, zero_row, 0)

    def zero_acc(i, _):
      pltpu.sync_copy(rows_v, acc.at[pl.ds(sid * rows_per_tile + i * C, C)])
      return 0

    lax.fori_loop(0, rows_per_tile // C, zero_acc, 0)
    plsc.subcore_barrier()

    def run(base, count):
      pltpu.sync_copy(src_hbm.at[pl.ds(base, count)],
                      src_v.at[pl.ds(0, count)])
      pltpu.sync_copy(dst_hbm.at[pl.ds(base, count)],
                      dst_v.at[pl.ds(0, count)])

      def body(j, _):
        pltpu.async_copy(g_hbm.at[src_v.at[j]], rows_v, sem).wait()
        pltpu.sync_copy(rows_v, acc.at[dst_v.at[j]], add=True)
        return 0

      lax.fori_loop(0, count, body, 0)

    @pl.when(cid == 0)
    def _():
      run(sid * ch0, ch0)

    @pl.when(cid == 1)
    def _():
      run(TILES_PER_SC * ch0 + sid * ch1, ch1)

    plsc.subcore_barrier()

    def write_out(i, _):
      sl = pl.ds(sid * rows_per_tile + i * C, C)
      pltpu.sync_copy(acc.at[sl], out_hbm.at[cid, sl])
      return 0

    lax.fori_loop(0, rows_per_tile // C, write_out, 0)

  return agg_kernel


def _tc1_body(x_ref, w_ref, degp_ref, b_ref, g_ref, self_ref, dinv_ref):
  h = jnp.dot(x_ref[...], w_ref[...], preferred_element_type=jnp.float32)
  deg = degp_ref[0] + degp_ref[1] + 1.0
  dinv = lax.rsqrt(deg)
  g = dinv * h
  g_ref[...] = g
  self_ref[...] = dinv * g + b_ref[...]
  dinv_ref[...] = dinv


def _tc2_body(p_ref, self_ref, dinv_ref, o_ref):
  o_ref[...] = dinv_ref[...] * (p_ref[0] + p_ref[1]) + self_ref[...]


def kernel(x, edge_index, W, b):
  n, d_in = x.shape
  d = W.shape[1]
  e = edge_index.shape[1]

  tot_ch = -(-e // (TILES_PER_SC * C))
  tot_ch = -(-tot_ch // 8) * 8
  e_pad = tot_ch * TILES_PER_SC * C
  ch0 = max(8, int(round(tot_ch * FRAC0 / 8)) * 8)
  ch1 = tot_ch - ch0
  n_pad = -(-n // (TILES_PER_SC * C)) * (TILES_PER_SC * C)
  rows_per_tile = n_pad // TILES_PER_SC

  ei = edge_index.astype(jnp.int32)
  pad = jnp.full((e_pad - e,), n, jnp.int32)
  src_f = jnp.concatenate([ei[0], pad]).reshape(tot_ch * TILES_PER_SC, C)
  dst_f = jnp.concatenate([ei[1], pad]).reshape(tot_ch * TILES_PER_SC, C)
  x_pad = jnp.pad(x, ((0, n_pad - n), (0, 0)))

  degp = _deg_kernel_fn(n_pad, ch0, ch1, rows_per_tile)(dst_f)
  degp3 = degp.reshape(2, n_pad, 1)

  br = 128
  grid = n_pad // br
  g, selfpart, dinv = pl.pallas_call(
      _tc1_body,
      grid=(grid,),
      in_specs=[
          pl.BlockSpec((br, d_in), lambda i: (i, 0)),
          pl.BlockSpec((d_in, d), lambda i: (0, 0)),
          pl.BlockSpec((2, br, 1), lambda i: (0, i, 0)),
          pl.BlockSpec((1, d), lambda i: (0, 0)),
      ],
      out_specs=[
          pl.BlockSpec((br, d), lambda i: (i, 0)),
          pl.BlockSpec((br, d), lambda i: (i, 0)),
          pl.BlockSpec((br, 1), lambda i: (i, 0)),
      ],
      out_shape=[
          jax.ShapeDtypeStruct((n_pad, d), jnp.float32),
          jax.ShapeDtypeStruct((n_pad, d), jnp.float32),
          jax.ShapeDtypeStruct((n_pad, 1), jnp.float32),
      ],
  )(x_pad, W, degp3, b.reshape(1, d))

  p = _agg_kernel_fn(n_pad, d, ch0, ch1, rows_per_tile)(src_f, dst_f, g)

  out = pl.pallas_call(
      _tc2_body,
      grid=(grid,),
      in_specs=[
          pl.BlockSpec((2, br, d), lambda i: (0, i, 0)),
          pl.BlockSpec((br, d), lambda i: (i, 0)),
          pl.BlockSpec((br, 1), lambda i: (i, 0)),
      ],
      out_specs=pl.BlockSpec((br, d), lambda i: (i, 0)),
      out_shape=jax.ShapeDtypeStruct((n_pad, d), jnp.float32),
  )(p, selfpart, dinv)

  return out[:n]


# speedup vs baseline: 2.5627x; 2.5627x over previous
import functools

import jax
import jax.numpy as jnp
from jax import lax
from jax.experimental import pallas as pl
from jax.experimental.pallas import tpu as pltpu, tpu_sc as plsc

NUM_TILES = 32
TILES_PER_SC = 16
C = 128
FRAC0 = 0.5


def _deg_kernel_fn(n_pad, ch0, ch1, rows_per_tile):
  mesh = plsc.VectorSubcoreMesh(core_axis_name="c", subcore_axis_name="s")
  chm = max(ch0, ch1)

  @functools.partial(
      pl.kernel,
      out_type=jax.ShapeDtypeStruct((2, n_pad), jnp.float32),
      mesh=mesh,
      scratch_types=[
          pltpu.VMEM((chm, C), jnp.int32),
          pltpu.VMEM((C,), jnp.float32),
          pltpu.VMEM((rows_per_tile,), jnp.float32),
          pltpu.VMEM_SHARED((n_pad,), jnp.float32),
      ],
  )
  def deg_kernel(dst_hbm, out_hbm, idx_v, ones_v, zbuf_v, acc):
    cid = lax.axis_index("c")
    sid = lax.axis_index("s")

    def set_ones(i, _):
      ones_v[pl.ds(i * 16, 16)] = jnp.ones((16,), jnp.float32)
      return 0

    lax.fori_loop(0, C // 16, set_ones, 0)

    def set_zero(i, _):
      zbuf_v[pl.ds(i * 16, 16)] = jnp.zeros((16,), jnp.float32)
      return 0

    lax.fori_loop(0, rows_per_tile // 16, set_zero, 0)
    pltpu.sync_copy(zbuf_v, acc.at[pl.ds(sid * rows_per_tile, rows_per_tile)])
    plsc.subcore_barrier()

    def run(base, count):
      pltpu.sync_copy(dst_hbm.at[pl.ds(base, count)],
                      idx_v.at[pl.ds(0, count)])

      def body(j, _):
        pltpu.sync_copy(ones_v, acc.at[idx_v.at[j]], add=True)
        return 0

      lax.fori_loop(0, count, body, 0)

    @pl.when(cid == 0)
    def _():
      run(sid * ch0, ch0)

    @pl.when(cid == 1)
    def _():
      run(TILES_PER_SC * ch0 + sid * ch1, ch1)

    plsc.subcore_barrier()
    sl = pl.ds(sid * rows_per_tile, rows_per_tile)
    pltpu.sync_copy(acc.at[sl], out_hbm.at[cid, sl])

  return deg_kernel


def _agg_kernel_fn(n_pad, d, ch0, ch1, rows_per_tile):
  mesh = plsc.VectorSubcoreMesh(core_axis_name="c", subcore_axis_name="s")
  chm = max(ch0, ch1)

  @functools.partial(
      pl.kernel,
      out_type=jax.ShapeDtypeStruct((2, n_pad, d), jnp.float32),
      mesh=mesh,
      scratch_types=[
          pltpu.VMEM((chm, C), jnp.int32),
          pltpu.VMEM((chm, C), jnp.int32),
          pltpu.VMEM((C, d), jnp.float32),
          pltpu.VMEM_SHARED((n_pad, d), jnp.float32),
          pltpu.SemaphoreType.DMA,
      ],
  )
  def agg_kernel(src_hbm, dst_hbm, g_hbm, out_hbm, src_v, dst_v, rows_v, acc,
                 sem):
    cid = lax.axis_index("c")
    sid = lax.axis_index("s")

    def zero_row(i, _):
      def zero_chunk(k, _2):
        rows_v[i, pl.ds(k * 16, 16)] = jnp.zeros((16,), jnp.float32)
        return 0

      lax.fori_loop(0, d // 16, zero_chunk, 0)
      return 0

    lax.fori_loop(0, C, zero_row, 0)

    def zero_acc(i, _):
      pltpu.sync_copy(rows_v, acc.at[pl.ds(sid * rows_per_tile + i * C, C)])
      return 0

    lax.fori_loop(0, rows_per_tile // C, zero_acc, 0)
    plsc.subcore_barrier()

    def run(base, count):
      pltpu.sync_copy(src_hbm.at[pl.ds(base, count)],
                      src_v.at[pl.ds(0, count)])
      pltpu.sync_copy(dst_hbm.at[pl.ds(base, count)],
                      dst_v.at[pl.ds(0, count)])

      def body(j, _):
        pltpu.async_copy(g_hbm.at[src_v.at[j]], rows_v, sem).wait()
        pltpu.sync_copy(rows_v, acc.at[dst_v.at[j]], add=True)
        return 0

      lax.fori_loop(0, count, body, 0)

    @pl.when(cid == 0)
    def _():
      run(sid * ch0, ch0)

    @pl.when(cid == 1)
    def _():
      run(TILES_PER_SC * ch0 + sid * ch1, ch1)

    plsc.subcore_barrier()

    def write_out(i, _):
      sl = pl.ds(sid * rows_per_tile + i * C, C)
      pltpu.sync_copy(acc.at[sl], out_hbm.at[cid, sl])
      return 0

    lax.fori_loop(0, rows_per_tile // C, write_out, 0)

  return agg_kernel


def _tc1_body(x_ref, w_ref, degp_ref, b_ref, g_ref, self_ref, dinv_ref):
  h = jnp.dot(x_ref[...], w_ref[...], preferred_element_type=jnp.float32)
  deg = degp_ref[0] + degp_ref[1] + 1.0
  dinv = lax.rsqrt(deg)
  g = dinv * h
  g_ref[...] = g
  self_ref[...] = dinv * g + b_ref[...]
  dinv_ref[...] = dinv


def _tc2_body(p_ref, self_ref, dinv_ref, o_ref):
  o_ref[...] = dinv_ref[...] * (p_ref[0] + p_ref[1]) + self_ref[...]


def kernel(x, edge_index, W, b):
  n, d_in = x.shape
  d = W.shape[1]
  e = edge_index.shape[1]

  tot_ch = -(-e // (TILES_PER_SC * C))
  tot_ch = -(-tot_ch // 16) * 16
  e_pad = tot_ch * TILES_PER_SC * C
  ch0 = max(8, int(round(tot_ch * FRAC0 / 8)) * 8)
  ch1 = tot_ch - ch0
  n_pad = -(-n // (TILES_PER_SC * C)) * (TILES_PER_SC * C)
  rows_per_tile = n_pad // TILES_PER_SC

  ei = edge_index.astype(jnp.int32)
  pad = (n + jnp.arange(e_pad - e, dtype=jnp.int32) % (n_pad - n)).astype(
      jnp.int32)
  src_f = jnp.concatenate([ei[0], pad]).reshape(tot_ch * TILES_PER_SC, C)
  dst_f = jnp.concatenate([ei[1], pad]).reshape(tot_ch * TILES_PER_SC, C)
  x_pad = jnp.pad(x, ((0, n_pad - n), (0, 0)))

  degp = _deg_kernel_fn(n_pad, ch0, ch1, rows_per_tile)(dst_f)
  degp3 = degp.reshape(2, n_pad, 1)

  br = 128
  grid = n_pad // br
  g, selfpart, dinv = pl.pallas_call(
      _tc1_body,
      grid=(grid,),
      in_specs=[
          pl.BlockSpec((br, d_in), lambda i: (i, 0)),
          pl.BlockSpec((d_in, d), lambda i: (0, 0)),
          pl.BlockSpec((2, br, 1), lambda i: (0, i, 0)),
          pl.BlockSpec((1, d), lambda i: (0, 0)),
      ],
      out_specs=[
          pl.BlockSpec((br, d), lambda i: (i, 0)),
          pl.BlockSpec((br, d), lambda i: (i, 0)),
          pl.BlockSpec((br, 1), lambda i: (i, 0)),
      ],
      out_shape=[
          jax.ShapeDtypeStruct((n_pad, d), jnp.float32),
          jax.ShapeDtypeStruct((n_pad, d), jnp.float32),
          jax.ShapeDtypeStruct((n_pad, 1), jnp.float32),
      ],
  )(x_pad, W, degp3, b.reshape(1, d))

  p = _agg_kernel_fn(n_pad, d, ch0, ch1, rows_per_tile)(src_f, dst_f, g)

  out = pl.pallas_call(
      _tc2_body,
      grid=(grid,),
      in_specs=[
          pl.BlockSpec((2, br, d), lambda i: (0, i, 0)),
          pl.BlockSpec((br, d), lambda i: (i, 0)),
          pl.BlockSpec((br, 1), lambda i: (i, 0)),
      ],
      out_specs=pl.BlockSpec((br, d), lambda i: (i, 0)),
      out_shape=jax.ShapeDtypeStruct((n_pad, d), jnp.float32),
  )(p, selfpart, dinv)

  return out[:n]


# speedup vs baseline: 2.9332x; 1.1446x over previous
import functools

import jax
import jax.numpy as jnp
from jax import lax
from jax.experimental import pallas as pl
from jax.experimental.pallas import tpu as pltpu, tpu_sc as plsc

NUM_TILES = 32
TILES_PER_SC = 16
C = 128
FRAC0 = 0.5


def _deg_kernel_fn(n_pad, ch0, ch1, rows_per_tile):
  mesh = plsc.VectorSubcoreMesh(core_axis_name="c", subcore_axis_name="s")
  chm = max(ch0, ch1)

  @functools.partial(
      pl.kernel,
      out_type=jax.ShapeDtypeStruct((2, n_pad), jnp.float32),
      mesh=mesh,
      scratch_types=[
          pltpu.VMEM((chm, C), jnp.int32),
          pltpu.VMEM((C,), jnp.float32),
          pltpu.VMEM((rows_per_tile,), jnp.float32),
          pltpu.VMEM_SHARED((n_pad,), jnp.float32),
      ],
  )
  def deg_kernel(dst_hbm, out_hbm, idx_v, ones_v, zbuf_v, acc):
    cid = lax.axis_index("c")
    sid = lax.axis_index("s")

    def set_ones(i, _):
      ones_v[pl.ds(i * 16, 16)] = jnp.ones((16,), jnp.float32)
      return 0

    lax.fori_loop(0, C // 16, set_ones, 0)

    def set_zero(i, _):
      zbuf_v[pl.ds(i * 16, 16)] = jnp.zeros((16,), jnp.float32)
      return 0

    lax.fori_loop(0, rows_per_tile // 16, set_zero, 0)
    pltpu.sync_copy(zbuf_v, acc.at[pl.ds(sid * rows_per_tile, rows_per_tile)])
    plsc.subcore_barrier()

    def run(base, count):
      pltpu.sync_copy(dst_hbm.at[pl.ds(base, count)],
                      idx_v.at[pl.ds(0, count)])

      def body(j, _):
        pltpu.sync_copy(ones_v, acc.at[idx_v.at[j]], add=True)
        return 0

      lax.fori_loop(0, count, body, 0)

    @pl.when(cid == 0)
    def _():
      run(sid * ch0, ch0)

    @pl.when(cid == 1)
    def _():
      run(TILES_PER_SC * ch0 + sid * ch1, ch1)

    plsc.subcore_barrier()
    sl = pl.ds(sid * rows_per_tile, rows_per_tile)
    pltpu.sync_copy(acc.at[sl], out_hbm.at[cid, sl])

  return deg_kernel


def _agg_kernel_fn(n_pad, d, ch0, ch1, rows_per_tile):
  mesh = plsc.VectorSubcoreMesh(core_axis_name="c", subcore_axis_name="s")
  chm = max(ch0, ch1)

  @functools.partial(
      pl.kernel,
      out_type=jax.ShapeDtypeStruct((2, n_pad, d), jnp.float32),
      mesh=mesh,
      scratch_types=[
          pltpu.VMEM((chm, C), jnp.int32),
          pltpu.VMEM((chm, C), jnp.int32),
          pltpu.VMEM((C, d), jnp.float32),
          pltpu.VMEM_SHARED((n_pad, d), jnp.float32),
          pltpu.SemaphoreType.DMA,
      ],
  )
  def agg_kernel(src_hbm, dst_hbm, g_hbm, out_hbm, src_v, dst_v, rows_v, acc,
                 sem):
    cid = lax.axis_index("c")
    sid = lax.axis_index("s")

    def zero_row(i, _):
      def zero_chunk(k, _2):
        rows_v[i, pl.ds(k * 16, 16)] = jnp.zeros((16,), jnp.float32)
        return 0

      lax.fori_loop(0, d // 16, zero_chunk, 0)
      return 0

    lax.fori_loop(0, C, zero_row, 0)

    def zero_acc(i, _):
      pltpu.sync_copy(rows_v, acc.at[pl.ds(sid * rows_per_tile + i * C, C)])
      return 0

    lax.fori_loop(0, rows_per_tile // C, zero_acc, 0)
    plsc.subcore_barrier()

    def run(base, count):
      pltpu.sync_copy(src_hbm.at[pl.ds(base, count)],
                      src_v.at[pl.ds(0, count)])
      pltpu.sync_copy(dst_hbm.at[pl.ds(base, count)],
                      dst_v.at[pl.ds(0, count)])

      def body(j, _):
        pltpu.async_copy(g_hbm.at[src_v.at[j]], rows_v, sem).wait()
        pltpu.sync_copy(rows_v, acc.at[dst_v.at[j]], add=True)
        return 0

      lax.fori_loop(0, count, body, 0)

    @pl.when(cid == 0)
    def _():
      run(sid * ch0, ch0)

    @pl.when(cid == 1)
    def _():
      run(TILES_PER_SC * ch0 + sid * ch1, ch1)

    plsc.subcore_barrier()

    def write_out(i, _):
      sl = pl.ds(sid * rows_per_tile + i * C, C)
      pltpu.sync_copy(acc.at[sl], out_hbm.at[cid, sl])
      return 0

    lax.fori_loop(0, rows_per_tile // C, write_out, 0)

  return agg_kernel


def _tc1_body(x_ref, w_ref, degp_ref, b_ref, g_ref, self_ref, dinv_ref):
  h = jnp.dot(x_ref[...], w_ref[...], preferred_element_type=jnp.float32)
  deg = degp_ref[0] + degp_ref[1] + 1.0
  dinv = lax.rsqrt(deg)
  g = dinv * h
  g_ref[...] = g
  self_ref[...] = dinv * g + b_ref[...]
  dinv_ref[...] = dinv


def _tc2_body(p_ref, self_ref, dinv_ref, o_ref):
  o_ref[...] = dinv_ref[...] * (p_ref[0] + p_ref[1]) + self_ref[...]


def kernel(x, edge_index, W, b):
  n, d_in = x.shape
  d = W.shape[1]
  e = edge_index.shape[1]

  tot_ch = -(-e // (TILES_PER_SC * C))
  tot_ch = -(-tot_ch // 16) * 16
  e_pad = tot_ch * TILES_PER_SC * C
  ch0 = max(8, int(round(tot_ch * FRAC0 / 8)) * 8)
  ch1 = tot_ch - ch0
  n_pad = -(-n // (TILES_PER_SC * C)) * (TILES_PER_SC * C)
  rows_per_tile = n_pad // TILES_PER_SC

  ei = edge_index.astype(jnp.int32)
  pad = (n + jnp.arange(e_pad - e, dtype=jnp.int32) % (n_pad - n)).astype(
      jnp.int32)
  src_f = jnp.concatenate([ei[0], pad]).reshape(tot_ch * TILES_PER_SC, C)
  dst_f = jnp.concatenate([ei[1], pad]).reshape(tot_ch * TILES_PER_SC, C)
  x_pad = jnp.pad(x, ((0, n_pad - n), (0, 0)))

  degp = _deg_kernel_fn(n_pad, ch0, ch1, rows_per_tile)(dst_f)
  degp3 = degp.reshape(2, n_pad, 1)

  br = 256
  grid = n_pad // br
  g, selfpart, dinv = pl.pallas_call(
      _tc1_body,
      grid=(grid,),
      in_specs=[
          pl.BlockSpec((br, d_in), lambda i: (i, 0)),
          pl.BlockSpec((d_in, d), lambda i: (0, 0)),
          pl.BlockSpec((2, br, 1), lambda i: (0, i, 0)),
          pl.BlockSpec((1, d), lambda i: (0, 0)),
      ],
      out_specs=[
          pl.BlockSpec((br, d), lambda i: (i, 0)),
          pl.BlockSpec((br, d), lambda i: (i, 0)),
          pl.BlockSpec((br, 1), lambda i: (i, 0)),
      ],
      out_shape=[
          jax.ShapeDtypeStruct((n_pad, d), jnp.float32),
          jax.ShapeDtypeStruct((n_pad, d), jnp.float32),
          jax.ShapeDtypeStruct((n_pad, 1), jnp.float32),
      ],
  )(x_pad, W, degp3, b.reshape(1, d))

  p = _agg_kernel_fn(n_pad, d, ch0, ch1, rows_per_tile)(src_f, dst_f, g)

  out = pl.pallas_call(
      _tc2_body,
      grid=(grid,),
      in_specs=[
          pl.BlockSpec((2, br, d), lambda i: (0, i, 0)),
          pl.BlockSpec((br, d), lambda i: (i, 0)),
          pl.BlockSpec((br, 1), lambda i: (i, 0)),
      ],
      out_specs=pl.BlockSpec((br, d), lambda i: (i, 0)),
      out_shape=jax.ShapeDtypeStruct((n_pad, d), jnp.float32),
  )(p, selfpart, dinv)

  return out[:n]


# speedup vs baseline: 3.1539x; 1.0753x over previous
import functools

import jax
import jax.numpy as jnp
from jax import lax
from jax.experimental import pallas as pl
from jax.experimental.pallas import tpu as pltpu, tpu_sc as plsc

NUM_TILES = 32
TILES_PER_SC = 16
C = 128
FRAC0 = 0.5


def _deg_kernel_fn(n_pad, ch0, ch1, rows_per_tile):
  mesh = plsc.VectorSubcoreMesh(core_axis_name="c", subcore_axis_name="s")
  chm = max(ch0, ch1)

  @functools.partial(
      pl.kernel,
      out_type=jax.ShapeDtypeStruct((2, n_pad), jnp.float32),
      mesh=mesh,
      scratch_types=[
          pltpu.VMEM((chm, C), jnp.int32),
          pltpu.VMEM((C,), jnp.float32),
          pltpu.VMEM((rows_per_tile,), jnp.float32),
          pltpu.VMEM_SHARED((n_pad,), jnp.float32),
      ],
  )
  def deg_kernel(dst_hbm, out_hbm, idx_v, ones_v, zbuf_v, acc):
    cid = lax.axis_index("c")
    sid = lax.axis_index("s")

    def set_ones(i, _):
      ones_v[pl.ds(i * 16, 16)] = jnp.ones((16,), jnp.float32)
      return 0

    lax.fori_loop(0, C // 16, set_ones, 0)

    def set_zero(i, _):
      zbuf_v[pl.ds(i * 16, 16)] = jnp.zeros((16,), jnp.float32)
      return 0

    lax.fori_loop(0, rows_per_tile // 16, set_zero, 0)
    pltpu.sync_copy(zbuf_v, acc.at[pl.ds(sid * rows_per_tile, rows_per_tile)])
    plsc.subcore_barrier()

    def run(base, count):
      pltpu.sync_copy(dst_hbm.at[pl.ds(base, count)],
                      idx_v.at[pl.ds(0, count)])

      def body(j, _):
        pltpu.sync_copy(ones_v, acc.at[idx_v.at[j]], add=True)
        return 0

      lax.fori_loop(0, count, body, 0)

    @pl.when(cid == 0)
    def _():
      run(sid * ch0, ch0)

    @pl.when(cid == 1)
    def _():
      run(TILES_PER_SC * ch0 + sid * ch1, ch1)

    plsc.subcore_barrier()
    sl = pl.ds(sid * rows_per_tile, rows_per_tile)
    pltpu.sync_copy(acc.at[sl], out_hbm.at[cid, sl])

  return deg_kernel


def _agg_kernel_fn(n_pad, d, ch0, ch1, rows_per_tile):
  mesh = plsc.VectorSubcoreMesh(core_axis_name="c", subcore_axis_name="s")
  chm = max(ch0, ch1)

  @functools.partial(
      pl.kernel,
      out_type=jax.ShapeDtypeStruct((2, n_pad, d), jnp.float32),
      mesh=mesh,
      scratch_types=[
          pltpu.VMEM((chm, C), jnp.int32),
          pltpu.VMEM((chm, C), jnp.int32),
          pltpu.VMEM((C, d), jnp.float32),
          pltpu.VMEM_SHARED((n_pad, d), jnp.float32),
          pltpu.SemaphoreType.DMA,
      ],
  )
  def agg_kernel(src_hbm, dst_hbm, g_hbm, out_hbm, src_v, dst_v, rows_v, acc,
                 sem):
    cid = lax.axis_index("c")
    sid = lax.axis_index("s")

    def zero_row(i, _):
      def zero_chunk(k, _2):
        rows_v[i, pl.ds(k * 16, 16)] = jnp.zeros((16,), jnp.float32)
        return 0

      lax.fori_loop(0, d // 16, zero_chunk, 0)
      return 0

    lax.fori_loop(0, C, zero_row, 0)

    def zero_acc(i, _):
      pltpu.sync_copy(rows_v, acc.at[pl.ds(sid * rows_per_tile + i * C, C)])
      return 0

    lax.fori_loop(0, rows_per_tile // C, zero_acc, 0)
    plsc.subcore_barrier()

    def run(base, count):
      pltpu.sync_copy(src_hbm.at[pl.ds(base, count)],
                      src_v.at[pl.ds(0, count)])
      pltpu.sync_copy(dst_hbm.at[pl.ds(base, count)],
                      dst_v.at[pl.ds(0, count)])

      def body(j, _):
        pltpu.async_copy(g_hbm.at[src_v.at[j]], rows_v, sem).wait()
        pltpu.sync_copy(rows_v, acc.at[dst_v.at[j]], add=True)
        return 0

      lax.fori_loop(0, count, body, 0)

    @pl.when(cid == 0)
    def _():
      run(sid * ch0, ch0)

    @pl.when(cid == 1)
    def _():
      run(TILES_PER_SC * ch0 + sid * ch1, ch1)

    plsc.subcore_barrier()

    def write_out(i, _):
      sl = pl.ds(sid * rows_per_tile + i * C, C)
      pltpu.sync_copy(acc.at[sl], out_hbm.at[cid, sl])
      return 0

    lax.fori_loop(0, rows_per_tile // C, write_out, 0)

  return agg_kernel


def _tc1_body(x_ref, w_ref, degp_ref, b_ref, g_ref, self_ref, dinv_ref):
  h = jnp.dot(x_ref[...], w_ref[...], preferred_element_type=jnp.float32)
  deg = degp_ref[0] + degp_ref[1] + 1.0
  dinv = lax.rsqrt(deg)
  g = dinv * h
  g_ref[...] = g
  self_ref[...] = dinv * g + b_ref[...]
  dinv_ref[...] = dinv


def _tc2_body(p_ref, self_ref, dinv_ref, o_ref):
  o_ref[...] = dinv_ref[...] * (p_ref[0] + p_ref[1]) + self_ref[...]


def kernel(x, edge_index, W, b):
  n, d_in = x.shape
  d = W.shape[1]
  e = edge_index.shape[1]

  tot_ch = -(-e // (TILES_PER_SC * C))
  tot_ch = -(-tot_ch // 16) * 16
  e_pad = tot_ch * TILES_PER_SC * C
  ch0 = max(8, int(round(tot_ch * FRAC0 / 8)) * 8)
  ch1 = tot_ch - ch0
  n_pad = -(-n // (TILES_PER_SC * C)) * (TILES_PER_SC * C)
  rows_per_tile = n_pad // TILES_PER_SC

  ei = edge_index.astype(jnp.int32)
  pad = (n + jnp.arange(e_pad - e, dtype=jnp.int32) % (n_pad - n)).astype(
      jnp.int32)
  src_f = jnp.concatenate([ei[0], pad]).reshape(tot_ch * TILES_PER_SC, C)
  dst_f = jnp.concatenate([ei[1], pad]).reshape(tot_ch * TILES_PER_SC, C)
  x_pad = jnp.pad(x, ((0, n_pad - n), (0, 0)))

  degp = _deg_kernel_fn(n_pad, ch0, ch1, rows_per_tile)(dst_f)
  degp3 = degp.reshape(2, n_pad, 1)

  br = 512
  grid = n_pad // br
  g, selfpart, dinv = pl.pallas_call(
      _tc1_body,
      grid=(grid,),
      in_specs=[
          pl.BlockSpec((br, d_in), lambda i: (i, 0)),
          pl.BlockSpec((d_in, d), lambda i: (0, 0)),
          pl.BlockSpec((2, br, 1), lambda i: (0, i, 0)),
          pl.BlockSpec((1, d), lambda i: (0, 0)),
      ],
      out_specs=[
          pl.BlockSpec((br, d), lambda i: (i, 0)),
          pl.BlockSpec((br, d), lambda i: (i, 0)),
          pl.BlockSpec((br, 1), lambda i: (i, 0)),
      ],
      out_shape=[
          jax.ShapeDtypeStruct((n_pad, d), jnp.float32),
          jax.ShapeDtypeStruct((n_pad, d), jnp.float32),
          jax.ShapeDtypeStruct((n_pad, 1), jnp.float32),
      ],
  )(x_pad, W, degp3, b.reshape(1, d))

  p = _agg_kernel_fn(n_pad, d, ch0, ch1, rows_per_tile)(src_f, dst_f, g)

  out = pl.pallas_call(
      _tc2_body,
      grid=(grid,),
      in_specs=[
          pl.BlockSpec((2, br, d), lambda i: (0, i, 0)),
          pl.BlockSpec((br, d), lambda i: (i, 0)),
          pl.BlockSpec((br, 1), lambda i: (i, 0)),
      ],
      out_specs=pl.BlockSpec((br, d), lambda i: (i, 0)),
      out_shape=jax.ShapeDtypeStruct((n_pad, d), jnp.float32),
  )(p, selfpart, dinv)

  return out[:n]


# speedup vs baseline: 3.2895x; 1.0430x over previous
import functools

import jax
import jax.numpy as jnp
from jax import lax
from jax.experimental import pallas as pl
from jax.experimental.pallas import tpu as pltpu, tpu_sc as plsc

NUM_TILES = 32
TILES_PER_SC = 16
C = 128
FRAC0 = 0.5


def _deg_kernel_fn(n_pad, ch0, ch1, rows_per_tile):
  mesh = plsc.VectorSubcoreMesh(core_axis_name="c", subcore_axis_name="s")
  chm = max(ch0, ch1)

  @functools.partial(
      pl.kernel,
      out_type=jax.ShapeDtypeStruct((2, n_pad), jnp.float32),
      mesh=mesh,
      scratch_types=[
          pltpu.VMEM((chm, C), jnp.int32),
          pltpu.VMEM((C,), jnp.float32),
          pltpu.VMEM((rows_per_tile,), jnp.float32),
          pltpu.VMEM_SHARED((n_pad,), jnp.float32),
      ],
  )
  def deg_kernel(dst_hbm, out_hbm, idx_v, ones_v, zbuf_v, acc):
    cid = lax.axis_index("c")
    sid = lax.axis_index("s")

    def set_ones(i, _):
      ones_v[pl.ds(i * 16, 16)] = jnp.ones((16,), jnp.float32)
      return 0

    lax.fori_loop(0, C // 16, set_ones, 0)

    def set_zero(i, _):
      zbuf_v[pl.ds(i * 16, 16)] = jnp.zeros((16,), jnp.float32)
      return 0

    lax.fori_loop(0, rows_per_tile // 16, set_zero, 0)
    pltpu.sync_copy(zbuf_v, acc.at[pl.ds(sid * rows_per_tile, rows_per_tile)])
    plsc.subcore_barrier()

    def run(base, count):
      pltpu.sync_copy(dst_hbm.at[pl.ds(base, count)],
                      idx_v.at[pl.ds(0, count)])

      def body(j, _):
        pltpu.sync_copy(ones_v, acc.at[idx_v.at[j]], add=True)
        return 0

      lax.fori_loop(0, count, body, 0)

    @pl.when(cid == 0)
    def _():
      run(sid * ch0, ch0)

    @pl.when(cid == 1)
    def _():
      run(TILES_PER_SC * ch0 + sid * ch1, ch1)

    plsc.subcore_barrier()
    sl = pl.ds(sid * rows_per_tile, rows_per_tile)
    pltpu.sync_copy(acc.at[sl], out_hbm.at[cid, sl])

  return deg_kernel


def _agg_kernel_fn(n_pad, d, ch0, ch1, rows_per_tile):
  mesh = plsc.VectorSubcoreMesh(core_axis_name="c", subcore_axis_name="s")
  chm = max(ch0, ch1)

  @functools.partial(
      pl.kernel,
      out_type=jax.ShapeDtypeStruct((2, n_pad, d), jnp.float32),
      mesh=mesh,
      scratch_types=[
          pltpu.VMEM((chm, C), jnp.int32),
          pltpu.VMEM((chm, C), jnp.int32),
          pltpu.VMEM((C, d), jnp.float32),
          pltpu.VMEM_SHARED((n_pad, d), jnp.float32),
          pltpu.SemaphoreType.DMA,
      ],
  )
  def agg_kernel(src_hbm, dst_hbm, g_hbm, out_hbm, src_v, dst_v, rows_v, acc,
                 sem):
    cid = lax.axis_index("c")
    sid = lax.axis_index("s")

    def zero_row(i, _):
      def zero_chunk(k, _2):
        rows_v[i, pl.ds(k * 16, 16)] = jnp.zeros((16,), jnp.float32)
        return 0

      lax.fori_loop(0, d // 16, zero_chunk, 0)
      return 0

    lax.fori_loop(0, C, zero_row, 0)

    def zero_acc(i, _):
      pltpu.sync_copy(rows_v, acc.at[pl.ds(sid * rows_per_tile + i * C, C)])
      return 0

    lax.fori_loop(0, rows_per_tile // C, zero_acc, 0)
    plsc.subcore_barrier()

    def run(base, count):
      pltpu.sync_copy(src_hbm.at[pl.ds(base, count)],
                      src_v.at[pl.ds(0, count)])
      pltpu.sync_copy(dst_hbm.at[pl.ds(base, count)],
                      dst_v.at[pl.ds(0, count)])

      def body(j, _):
        pltpu.async_copy(g_hbm.at[src_v.at[j]], rows_v, sem).wait()
        pltpu.sync_copy(rows_v, acc.at[dst_v.at[j]], add=True)
        return 0

      lax.fori_loop(0, count, body, 0)

    @pl.when(cid == 0)
    def _():
      run(sid * ch0, ch0)

    @pl.when(cid == 1)
    def _():
      run(TILES_PER_SC * ch0 + sid * ch1, ch1)

    plsc.subcore_barrier()

    def write_out(i, _):
      sl = pl.ds(sid * rows_per_tile + i * C, C)
      pltpu.sync_copy(acc.at[sl], out_hbm.at[cid, sl])
      return 0

    lax.fori_loop(0, rows_per_tile // C, write_out, 0)

  return agg_kernel


def _tc1_body(x_ref, w_ref, degp_ref, b_ref, g_ref, self_ref, dinv_ref):
  h = jnp.dot(x_ref[...], w_ref[...], preferred_element_type=jnp.float32)
  deg = degp_ref[0] + degp_ref[1] + 1.0
  dinv = lax.rsqrt(deg)
  g = dinv * h
  g_ref[...] = g
  self_ref[...] = dinv * g + b_ref[...]
  dinv_ref[...] = dinv


def _tc2_body(p_ref, self_ref, dinv_ref, o_ref):
  o_ref[...] = dinv_ref[...] * (p_ref[0] + p_ref[1]) + self_ref[...]


def kernel(x, edge_index, W, b):
  n, d_in = x.shape
  d = W.shape[1]
  e = edge_index.shape[1]

  tot_ch = -(-e // (TILES_PER_SC * C))
  tot_ch = -(-tot_ch // 16) * 16
  e_pad = tot_ch * TILES_PER_SC * C
  ch0 = max(8, int(round(tot_ch * FRAC0 / 8)) * 8)
  ch1 = tot_ch - ch0
  n_pad = -(-n // (TILES_PER_SC * C)) * (TILES_PER_SC * C)
  rows_per_tile = n_pad // TILES_PER_SC

  ei = edge_index.astype(jnp.int32)
  pad = (n + jnp.arange(e_pad - e, dtype=jnp.int32) % (n_pad - n)).astype(
      jnp.int32)
  src_f = jnp.concatenate([ei[0], pad]).reshape(tot_ch * TILES_PER_SC, C)
  dst_f = jnp.concatenate([ei[1], pad]).reshape(tot_ch * TILES_PER_SC, C)
  x_pad = jnp.pad(x, ((0, n_pad - n), (0, 0)))

  degp = _deg_kernel_fn(n_pad, ch0, ch1, rows_per_tile)(dst_f)
  degp3 = degp.reshape(2, n_pad, 1)

  br = 1024
  grid = n_pad // br
  g, selfpart, dinv = pl.pallas_call(
      _tc1_body,
      grid=(grid,),
      in_specs=[
          pl.BlockSpec((br, d_in), lambda i: (i, 0)),
          pl.BlockSpec((d_in, d), lambda i: (0, 0)),
          pl.BlockSpec((2, br, 1), lambda i: (0, i, 0)),
          pl.BlockSpec((1, d), lambda i: (0, 0)),
      ],
      out_specs=[
          pl.BlockSpec((br, d), lambda i: (i, 0)),
          pl.BlockSpec((br, d), lambda i: (i, 0)),
          pl.BlockSpec((br, 1), lambda i: (i, 0)),
      ],
      out_shape=[
          jax.ShapeDtypeStruct((n_pad, d), jnp.float32),
          jax.ShapeDtypeStruct((n_pad, d), jnp.float32),
          jax.ShapeDtypeStruct((n_pad, 1), jnp.float32),
      ],
  )(x_pad, W, degp3, b.reshape(1, d))

  p = _agg_kernel_fn(n_pad, d, ch0, ch1, rows_per_tile)(src_f, dst_f, g)

  out = pl.pallas_call(
      _tc2_body,
      grid=(grid,),
      in_specs=[
          pl.BlockSpec((2, br, d), lambda i: (0, i, 0)),
          pl.BlockSpec((br, d), lambda i: (i, 0)),
          pl.BlockSpec((br, 1), lambda i: (i, 0)),
      ],
      out_specs=pl.BlockSpec((br, d), lambda i: (i, 0)),
      out_shape=jax.ShapeDtypeStruct((n_pad, d), jnp.float32),
  )(p, selfpart, dinv)

  return out[:n]


# speedup vs baseline: 3.3385x; 1.0149x over previous
import functools

import jax
import jax.numpy as jnp
from jax import lax
from jax.experimental import pallas as pl
from jax.experimental.pallas import tpu as pltpu, tpu_sc as plsc

NUM_TILES = 32
TILES_PER_SC = 16
C = 128
FRAC0 = 0.5


def _deg_kernel_fn(n_pad, ch0, ch1, rows_per_tile):
  mesh = plsc.VectorSubcoreMesh(core_axis_name="c", subcore_axis_name="s")
  chm = max(ch0, ch1)

  @functools.partial(
      pl.kernel,
      out_type=jax.ShapeDtypeStruct((2, n_pad), jnp.float32),
      mesh=mesh,
      scratch_types=[
          pltpu.VMEM((chm, C), jnp.int32),
          pltpu.VMEM((C,), jnp.float32),
          pltpu.VMEM((rows_per_tile,), jnp.float32),
          pltpu.VMEM_SHARED((n_pad,), jnp.float32),
      ],
  )
  def deg_kernel(dst_hbm, out_hbm, idx_v, ones_v, zbuf_v, acc):
    cid = lax.axis_index("c")
    sid = lax.axis_index("s")

    def set_ones(i, _):
      ones_v[pl.ds(i * 16, 16)] = jnp.ones((16,), jnp.float32)
      return 0

    lax.fori_loop(0, C // 16, set_ones, 0)

    def set_zero(i, _):
      zbuf_v[pl.ds(i * 16, 16)] = jnp.zeros((16,), jnp.float32)
      return 0

    lax.fori_loop(0, rows_per_tile // 16, set_zero, 0)
    pltpu.sync_copy(zbuf_v, acc.at[pl.ds(sid * rows_per_tile, rows_per_tile)])
    plsc.subcore_barrier()

    def run(base, count):
      pltpu.sync_copy(dst_hbm.at[pl.ds(base, count)],
                      idx_v.at[pl.ds(0, count)])

      def body(j, _):
        pltpu.sync_copy(ones_v, acc.at[idx_v.at[j]], add=True)
        return 0

      lax.fori_loop(0, count, body, 0)

    @pl.when(cid == 0)
    def _():
      run(sid * ch0, ch0)

    @pl.when(cid == 1)
    def _():
      run(TILES_PER_SC * ch0 + sid * ch1, ch1)

    plsc.subcore_barrier()
    sl = pl.ds(sid * rows_per_tile, rows_per_tile)
    pltpu.sync_copy(acc.at[sl], out_hbm.at[cid, sl])

  return deg_kernel


def _agg_kernel_fn(n_pad, d, ch0, ch1, rows_per_tile):
  mesh = plsc.VectorSubcoreMesh(core_axis_name="c", subcore_axis_name="s")
  chm = max(ch0, ch1)

  @functools.partial(
      pl.kernel,
      out_type=jax.ShapeDtypeStruct((2, n_pad, d), jnp.float32),
      mesh=mesh,
      scratch_types=[
          pltpu.VMEM((chm, C), jnp.int32),
          pltpu.VMEM((chm, C), jnp.int32),
          pltpu.VMEM((C, d), jnp.float32),
          pltpu.VMEM_SHARED((n_pad, d), jnp.float32),
          pltpu.SemaphoreType.DMA,
      ],
  )
  def agg_kernel(src_hbm, dst_hbm, g_hbm, out_hbm, src_v, dst_v, rows_v, acc,
                 sem):
    cid = lax.axis_index("c")
    sid = lax.axis_index("s")

    def zero_row(i, _):
      def zero_chunk(k, _2):
        rows_v[i, pl.ds(k * 16, 16)] = jnp.zeros((16,), jnp.float32)
        return 0

      lax.fori_loop(0, d // 16, zero_chunk, 0)
      return 0

    lax.fori_loop(0, C, zero_row, 0)

    def zero_acc(i, _):
      pltpu.sync_copy(rows_v, acc.at[pl.ds(sid * rows_per_tile + i * C, C)])
      return 0

    lax.fori_loop(0, rows_per_tile // C, zero_acc, 0)
    plsc.subcore_barrier()

    def run(base, count):
      pltpu.sync_copy(src_hbm.at[pl.ds(base, count)],
                      src_v.at[pl.ds(0, count)])
      pltpu.sync_copy(dst_hbm.at[pl.ds(base, count)],
                      dst_v.at[pl.ds(0, count)])

      def body(j, _):
        pltpu.async_copy(g_hbm.at[src_v.at[j]], rows_v, sem).wait()
        pltpu.sync_copy(rows_v, acc.at[dst_v.at[j]], add=True)
        return 0

      lax.fori_loop(0, count, body, 0)

    @pl.when(cid == 0)
    def _():
      run(sid * ch0, ch0)

    @pl.when(cid == 1)
    def _():
      run(TILES_PER_SC * ch0 + sid * ch1, ch1)

    plsc.subcore_barrier()

    def write_out(i, _):
      sl = pl.ds(sid * rows_per_tile + i * C, C)
      pltpu.sync_copy(acc.at[sl], out_hbm.at[cid, sl])
      return 0

    lax.fori_loop(0, rows_per_tile // C, write_out, 0)

  return agg_kernel


def _tc1_body(x_ref, w_ref, degp_ref, b_ref, g_ref, self_ref, dinv_ref):
  h = jnp.dot(x_ref[...], w_ref[...], preferred_element_type=jnp.float32)
  deg = degp_ref[0] + degp_ref[1] + 1.0
  dinv = lax.rsqrt(deg)
  g = dinv * h
  g_ref[...] = g
  self_ref[...] = dinv * g + b_ref[...]
  dinv_ref[...] = dinv


def _tc2_body(p_ref, self_ref, dinv_ref, o_ref):
  o_ref[...] = dinv_ref[...] * (p_ref[0] + p_ref[1]) + self_ref[...]


def kernel(x, edge_index, W, b):
  n, d_in = x.shape
  d = W.shape[1]
  e = edge_index.shape[1]

  tot_ch = -(-e // (TILES_PER_SC * C))
  tot_ch = -(-tot_ch // 16) * 16
  e_pad = tot_ch * TILES_PER_SC * C
  ch0 = max(8, int(round(tot_ch * FRAC0 / 8)) * 8)
  ch1 = tot_ch - ch0
  n_pad = -(-n // (TILES_PER_SC * C)) * (TILES_PER_SC * C)
  rows_per_tile = n_pad // TILES_PER_SC

  ei = edge_index.astype(jnp.int32)
  pad = (n + jnp.arange(e_pad - e, dtype=jnp.int32) % (n_pad - n)).astype(
      jnp.int32)
  src_f = jnp.concatenate([ei[0], pad]).reshape(tot_ch * TILES_PER_SC, C)
  dst_f = jnp.concatenate([ei[1], pad]).reshape(tot_ch * TILES_PER_SC, C)
  x_pad = jnp.pad(x, ((0, n_pad - n), (0, 0)))

  degp = _deg_kernel_fn(n_pad, ch0, ch1, rows_per_tile)(dst_f)
  degp3 = degp.reshape(2, n_pad, 1)

  br = 2048
  grid = n_pad // br
  g, selfpart, dinv = pl.pallas_call(
      _tc1_body,
      grid=(grid,),
      in_specs=[
          pl.BlockSpec((br, d_in), lambda i: (i, 0)),
          pl.BlockSpec((d_in, d), lambda i: (0, 0)),
          pl.BlockSpec((2, br, 1), lambda i: (0, i, 0)),
          pl.BlockSpec((1, d), lambda i: (0, 0)),
      ],
      out_specs=[
          pl.BlockSpec((br, d), lambda i: (i, 0)),
          pl.BlockSpec((br, d), lambda i: (i, 0)),
          pl.BlockSpec((br, 1), lambda i: (i, 0)),
      ],
      out_shape=[
          jax.ShapeDtypeStruct((n_pad, d), jnp.float32),
          jax.ShapeDtypeStruct((n_pad, d), jnp.float32),
          jax.ShapeDtypeStruct((n_pad, 1), jnp.float32),
      ],
  )(x_pad, W, degp3, b.reshape(1, d))

  p = _agg_kernel_fn(n_pad, d, ch0, ch1, rows_per_tile)(src_f, dst_f, g)

  out = pl.pallas_call(
      _tc2_body,
      grid=(grid,),
      in_specs=[
          pl.BlockSpec((2, br, d), lambda i: (0, i, 0)),
          pl.BlockSpec((br, d), lambda i: (i, 0)),
          pl.BlockSpec((br, 1), lambda i: (i, 0)),
      ],
      out_specs=pl.BlockSpec((br, d), lambda i: (i, 0)),
      out_shape=jax.ShapeDtypeStruct((n_pad, d), jnp.float32),
  )(p, selfpart, dinv)

  return out[:n]
